# baseline (speedup 1.0000x reference)
import functools

import jax
import jax.numpy as jnp
from jax import lax
from jax.experimental import pallas as pl
from jax.experimental.pallas import tpu as pltpu
from jax.experimental.pallas import tpu_sc as plsc

B, T = 4, 32
C, H, W = 96, 56, 56
HW = H * W
N = C * HW
BT = B * T
K = 8
ROWS_PER_STEP = 16
G = 8


def _scores_body(x_ref, o_ref):
    s = jnp.sum(jnp.abs(x_ref[...]), axis=(1, 2)) * (1.0 / N)
    o_ref[...] = s.reshape(1, 1, ROWS_PER_STEP)


def _sc_mask_body(s_hbm, mask_hbm, src_hbm, sv, mv, fv, sem):
    cid = lax.axis_index('c')
    sid = lax.axis_index('s')

    @pl.when(jnp.logical_and(cid == 0, sid == 0))
    def _():
        pltpu.async_copy(s_hbm, sv, sem).wait()
        lanes = lax.iota(jnp.int32, 16)
        neg = jnp.float32(-jnp.inf)
        big = jnp.int32(2 * T)

        rots = [((lanes + sh) & 15).reshape(16, 1) for sh in (1, 2, 4, 8)]
        dnums = lax.GatherDimensionNumbers(
            offset_dims=(), collapsed_slice_dims=(0,), start_index_map=(0,))

        def rot(v, r):
            return lax.gather(v, r, dnums, slice_sizes=(1,),
                              mode=lax.GatherScatterMode.PROMISE_IN_BOUNDS)

        def allmax(v):
            for r in rots:
                v = jnp.maximum(v, rot(v, r))
            return v

        def allmin_i(v):
            for r in rots:
                v = jnp.minimum(v, rot(v, r))
            return v

        sel_chunks = []
        for b in range(B):
            wa = sv[pl.ds(b * T, 16)]
            wb = sv[pl.ds(b * T + 16, 16)]
            sela = jnp.zeros((16,), jnp.int32)
            selb = jnp.zeros((16,), jnp.int32)
            for _ in range(K):
                mx = allmax(jnp.maximum(wa, wb))
                ia = jnp.where(wa == mx, lanes, big)
                ib = jnp.where(wb == mx, lanes + 16, big)
                chosen = allmin_i(jnp.minimum(ia, ib))
                hit_a = lanes == chosen
                hit_b = (lanes + 16) == chosen
                sela = jnp.where(hit_a, 1, sela)
                selb = jnp.where(hit_b, 1, selb)
                wa = jnp.where(hit_a, neg, wa)
                wb = jnp.where(hit_b, neg, wb)
            mv[pl.ds(b * T, 16)] = sela
            mv[pl.ds(b * T + 16, 16)] = selb
            sel_chunks.append(sela)
            sel_chunks.append(selb)
        pltpu.async_copy(mv, mask_hbm, sem).wait()

        NCH = BT // 16
        neg1 = jnp.full((16,), -1, jnp.int32)
        bigi = jnp.full((16,), BT, jnp.int32)
        r8 = rots[3]
        marked = [jnp.where(sel_chunks[c] == 1, lanes + 16 * c, -1)
                  for c in range(NCH)]
        cur = [jnp.where(lanes >= 8, rot(marked[c], r8),
                         rot(marked[c - 1] if c > 0 else neg1, r8))
               for c in range(NCH)]
        cur = [jnp.maximum(marked[c], cur[c]) for c in range(NCH)]
        for stride_chunks in (1, 2, 4):
            cur = [jnp.maximum(cur[c], cur[c - stride_chunks])
                   if c >= stride_chunks else cur[c] for c in range(NCH)]
        m1 = bigi
        for c in range(NCH):
            m1 = jnp.minimum(m1, jnp.where(sel_chunks[c] == 1,
                                           lanes + 16 * c, BT))
        m2 = jnp.minimum(m1, rot(m1, r8))
        m2 = jnp.where(m2 >= BT, 0, m2)
        for c in range(NCH):
            fv[pl.ds(c * 16, 16)] = jnp.where(cur[c] < 0, m2, cur[c])
        pltpu.async_copy(fv, src_hbm, sem).wait()


_sc_mask = functools.partial(
    pl.kernel,
    mesh=plsc.VectorSubcoreMesh(core_axis_name='c', subcore_axis_name='s'),
    out_type=(jax.ShapeDtypeStruct((BT,), jnp.int32),
              jax.ShapeDtypeStruct((BT,), jnp.int32)),
    scratch_types=[
        pltpu.VMEM((BT,), jnp.float32),
        pltpu.VMEM((BT,), jnp.int32),
        pltpu.VMEM((BT,), jnp.int32),
        pltpu.SemaphoreType.DMA,
    ],
)(_sc_mask_body)


def _write_body(src_ref, keep_ref, *refs):
    g = pl.program_id(0)
    x_refs = refs[:G]
    o_ref = refs[G]
    for s in range(G):
        @pl.when(keep_ref[g * G + s] == 1)
        def _(s=s):
            o_ref[s, :, :] = x_refs[s][0, :, :]

        @pl.when(keep_ref[g * G + s] == 0)
        def _(s=s):
            o_ref[s, :, :] = jnp.zeros((HW, C), jnp.float32)


@jax.jit
def kernel(x):
    xv = x.transpose(0, 1, 3, 4, 2).reshape(BT, HW, C)

    scores_col = pl.pallas_call(
        _scores_body,
        grid=(BT // ROWS_PER_STEP,),
        in_specs=[pl.BlockSpec((ROWS_PER_STEP, HW, C), lambda i: (i, 0, 0))],
        out_specs=pl.BlockSpec((1, 1, ROWS_PER_STEP), lambda i: (i, 0, 0)),
        out_shape=jax.ShapeDtypeStruct((BT // ROWS_PER_STEP, 1, ROWS_PER_STEP),
                                       jnp.float32),
    )(xv)

    scores_flat = scores_col.reshape(BT)

    mask_flat, src_flat = _sc_mask(scores_flat)
    mask_i32 = mask_flat.reshape(B, T)

    outv = pl.pallas_call(
        _write_body,
        grid_spec=pltpu.PrefetchScalarGridSpec(
            num_scalar_prefetch=2,
            grid=(BT // G,),
            in_specs=[
                pl.BlockSpec((1, HW, C),
                             lambda g, src_r, keep_r, s=s: (src_r[g * G + s],
                                                            0, 0))
                for s in range(G)
            ],
            out_specs=pl.BlockSpec((G, HW, C),
                                   lambda g, src_r, keep_r: (g, 0, 0)),
        ),
        out_shape=jax.ShapeDtypeStruct((BT, HW, C), jnp.float32),
    )(src_flat, mask_flat, *([xv] * G))

    masked = outv.reshape(B, T, H, W, C).transpose(0, 1, 4, 2, 3)
    return masked, mask_i32.astype(jnp.bool_)
